# baseline (speedup 1.0000x reference)
import jax
import jax.numpy as jnp
from jax.experimental import pallas as pl
from jax.experimental.pallas import tpu as pltpu

MAX_LEN = 50
BATCH = 1024
N_RULES = 256
N_TOKENS = 1000
COPY_LEN = 200

_PLANS = (
    (N_TOKENS, BATCH, 5),
    (BATCH, N_RULES, 2),
    (COPY_LEN, BATCH, 1),
)
_N_DMAS = sum(p[2] for p in _PLANS)


def _gather_body(s_ref, r_in, t_in, c_in, r_out, t_out, c_out,
                 r_buf, t_buf, c_buf, in_sems, out_sems):
    idx = (s_ref[0] - 1) % MAX_LEN

    ins = []
    outs = []
    q = 0
    for (src, dst, buf), (rows, _, k) in zip(
        ((t_in, t_out, t_buf), (r_in, r_out, r_buf), (c_in, c_out, c_buf)),
        _PLANS,
    ):
        ch = rows // k
        for j in range(k):
            sl = pl.ds(j * ch, ch)
            ins.append(
                pltpu.make_async_copy(src.at[idx, sl], buf.at[sl], in_sems.at[q])
            )
            outs.append(
                pltpu.make_async_copy(buf.at[sl], dst.at[sl], out_sems.at[q])
            )
            q += 1
    for c in ins:
        c.start()
    for cin, cout in zip(ins, outs):
        cin.wait()
        cout.start()
    for cout in outs:
        cout.wait()


def kernel(rule_prob, token_prob, copy_prob, length):
    token_t = jnp.swapaxes(token_prob, 1, 2)
    copy_t = jnp.swapaxes(copy_prob, 1, 2)

    grid_spec = pltpu.PrefetchScalarGridSpec(
        num_scalar_prefetch=1,
        grid=(1,),
        in_specs=[pl.BlockSpec(memory_space=pl.ANY)] * 3,
        out_specs=[pl.BlockSpec(memory_space=pl.ANY)] * 3,
        scratch_shapes=[
            pltpu.VMEM((BATCH, N_RULES), jnp.float32),
            pltpu.VMEM((N_TOKENS, BATCH), jnp.float32),
            pltpu.VMEM((COPY_LEN, BATCH), jnp.float32),
            pltpu.SemaphoreType.DMA((_N_DMAS,)),
            pltpu.SemaphoreType.DMA((_N_DMAS,)),
        ],
    )
    out_shape = [
        jax.ShapeDtypeStruct((BATCH, N_RULES), jnp.float32),
        jax.ShapeDtypeStruct((N_TOKENS, BATCH), jnp.float32),
        jax.ShapeDtypeStruct((COPY_LEN, BATCH), jnp.float32),
    ]
    r, t, c = pl.pallas_call(
        _gather_body, grid_spec=grid_spec, out_shape=out_shape
    )(length, rule_prob, token_t, copy_t)
    return (r, t.T, c.T)
